# Initial kernel scaffold; baseline (speedup 1.0000x reference)
import jax
import jax.numpy as jnp
from jax.experimental import pallas as pl


def kernel(x, edge_index, W1, b1, W2, b2, Wc, bc):
    raise NotImplementedError("write your pallas kernel here")


# speedup vs baseline: 7.5217x; 7.5217x over previous
import functools

import jax
import jax.numpy as jnp
from jax import lax
from jax.experimental import pallas as pl
from jax.experimental.pallas import tpu as pltpu
from jax.experimental.pallas import tpu_sc as plsc

N = 10000
E = 320000
D = 128
NC = 2
NS = 16
L = 16
NTILES = NC * NS
CHUNK = 128
NCHUNK = 80
EPT = CHUNK * NCHUNK
E_PAD = EPT * NTILES
N_PAD = 10240
RPT = N_PAD // NS

BLK = 640
GRID = N_PAD // BLK


def _mesh():
    return plsc.VectorSubcoreMesh(core_axis_name="c", subcore_axis_name="s")


@functools.partial(
    pl.kernel,
    out_type=jax.ShapeDtypeStruct((NC * N_PAD, L), jnp.float32),
    mesh=_mesh(),
    scratch_types=[
        pltpu.VMEM((CHUNK,), jnp.int32),
        pltpu.VMEM((CHUNK, L), jnp.float32),
        pltpu.VMEM((CHUNK, L), jnp.float32),
        pltpu.VMEM_SHARED((N_PAD, L), jnp.float32),
    ],
)
def _deg_kernel(dst_hbm, out_hbm, dcur_v, ones_v, zeros_v, acc_sh):
    c = lax.axis_index("c")
    s = lax.axis_index("s")
    w = c * NS + s

    def fill(i, _):
        ones_v[i, :] = jnp.ones((L,), jnp.float32)
        zeros_v[i, :] = jnp.zeros((L,), jnp.float32)
        return 0

    lax.fori_loop(0, CHUNK, fill, 0)

    zbase = s * RPT
    for b in range(RPT // CHUNK):
        pltpu.sync_copy(zeros_v, acc_sh.at[pl.ds(zbase + b * CHUNK, CHUNK)])
    plsc.subcore_barrier()

    def body(j, _):
        pltpu.sync_copy(dst_hbm.at[w * NCHUNK + j], dcur_v)
        pltpu.sync_copy(ones_v, acc_sh.at[dcur_v], add=True)
        return 0

    lax.fori_loop(0, NCHUNK, body, 0)
    plsc.subcore_barrier()

    pltpu.sync_copy(
        acc_sh.at[pl.ds(s * RPT, RPT)],
        out_hbm.at[pl.ds(c * N_PAD + s * RPT, RPT)],
    )


@functools.partial(
    pl.kernel,
    out_type=jax.ShapeDtypeStruct((NC * N_PAD, D), jnp.float32),
    mesh=_mesh(),
    scratch_types=[
        pltpu.VMEM((NCHUNK, CHUNK), jnp.int32),
        pltpu.VMEM((CHUNK,), jnp.int32),
        pltpu.VMEM((CHUNK, D), jnp.float32),
        pltpu.VMEM_SHARED((N_PAD, D), jnp.float32),
        pltpu.SemaphoreType.DMA,
    ],
)
def _agg_kernel(hp_hbm, src_hbm, dst_hbm, out_hbm, src_v, dcur_v,
                rows_v, acc_sh, sem):
    c = lax.axis_index("c")
    s = lax.axis_index("s")
    w = c * NS + s

    def zr(i, _):
        for jj in range(D // L):
            rows_v[i, pl.ds(jj * L, L)] = jnp.zeros((L,), jnp.float32)
        return 0

    lax.fori_loop(0, CHUNK, zr, 0)

    zbase = s * RPT
    for b in range(RPT // CHUNK):
        pltpu.sync_copy(rows_v, acc_sh.at[pl.ds(zbase + b * CHUNK, CHUNK)])
    plsc.subcore_barrier()

    pltpu.sync_copy(src_hbm.at[w], src_v)

    def body(j, _):
        pltpu.sync_copy(dst_hbm.at[w * NCHUNK + j], dcur_v)
        pltpu.async_copy(hp_hbm.at[src_v.at[j]], rows_v, sem).wait()
        pltpu.sync_copy(rows_v, acc_sh.at[dcur_v], add=True)
        return 0

    lax.fori_loop(0, NCHUNK, body, 0)
    plsc.subcore_barrier()

    pltpu.sync_copy(
        acc_sh.at[pl.ds(s * RPT, RPT)],
        out_hbm.at[pl.ds(c * N_PAD + s * RPT, RPT)],
    )


def _mm_body(x_ref, w_ref, o_ref):
    o_ref[...] = jnp.dot(x_ref[...], w_ref[...],
                         preferred_element_type=jnp.float32)


def _matmul(x, w):
    return pl.pallas_call(
        _mm_body,
        grid=(GRID,),
        in_specs=[
            pl.BlockSpec((BLK, D), lambda i: (i, 0)),
            pl.BlockSpec((D, D), lambda i: (0, 0)),
        ],
        out_specs=pl.BlockSpec((BLK, D), lambda i: (i, 0)),
        out_shape=jax.ShapeDtypeStruct((N_PAD, D), jnp.float32),
    )(x, w)


def _scale_body(d_ref, u_ref, hp_ref, inv_ref):
    cnt = d_ref[0, :, 0:1] + d_ref[1, :, 0:1]
    inv = lax.rsqrt(cnt + 1.0)
    inv_ref[...] = inv
    hp_ref[...] = u_ref[...] * inv


def _scale(degp, u):
    return pl.pallas_call(
        _scale_body,
        grid=(GRID,),
        in_specs=[
            pl.BlockSpec((NC, BLK, L), lambda i: (0, i, 0)),
            pl.BlockSpec((BLK, D), lambda i: (i, 0)),
        ],
        out_specs=[
            pl.BlockSpec((BLK, D), lambda i: (i, 0)),
            pl.BlockSpec((BLK, 1), lambda i: (i, 0)),
        ],
        out_shape=[
            jax.ShapeDtypeStruct((N_PAD, D), jnp.float32),
            jax.ShapeDtypeStruct((N_PAD, 1), jnp.float32),
        ],
    )(degp, u)


def _row_mask(i):
    ridx = lax.broadcasted_iota(jnp.int32, (BLK, 1), 0) + i * BLK
    return ridx < N


def _mid_body(s_ref, hp_ref, inv_ref, b_ref, w_ref, o_ref):
    inv = inv_ref[...]
    t = (s_ref[0] + s_ref[1] + hp_ref[...]) * inv + b_ref[...]
    h = jnp.maximum(t, 0.0)
    h = jnp.where(_row_mask(pl.program_id(0)), h, 0.0)
    o_ref[...] = jnp.dot(h, w_ref[...],
                         preferred_element_type=jnp.float32) * inv


def _mid(s, hp, inv, b, w):
    return pl.pallas_call(
        _mid_body,
        grid=(GRID,),
        in_specs=[
            pl.BlockSpec((NC, BLK, D), lambda i: (0, i, 0)),
            pl.BlockSpec((BLK, D), lambda i: (i, 0)),
            pl.BlockSpec((BLK, 1), lambda i: (i, 0)),
            pl.BlockSpec((1, D), lambda i: (0, 0)),
            pl.BlockSpec((D, D), lambda i: (0, 0)),
        ],
        out_specs=pl.BlockSpec((BLK, D), lambda i: (i, 0)),
        out_shape=jax.ShapeDtypeStruct((N_PAD, D), jnp.float32),
    )(s, hp, inv, b, w)


def _fin_body(s_ref, hp_ref, inv_ref, b_ref, wc_ref, bc_ref, o_ref, acc_ref):
    i = pl.program_id(0)

    @pl.when(i == 0)
    def _():
        acc_ref[...] = jnp.zeros_like(acc_ref)

    t = (s_ref[0] + s_ref[1] + hp_ref[...]) * inv_ref[...] + b_ref[...]
    h = jnp.maximum(t, 0.0)
    h = jnp.where(_row_mask(i), h, 0.0)
    acc_ref[...] += jnp.sum(h, axis=0, keepdims=True)

    @pl.when(i == pl.num_programs(0) - 1)
    def _():
        g = acc_ref[...] * (1.0 / N)
        logits = jnp.dot(g, wc_ref[...],
                         preferred_element_type=jnp.float32) + bc_ref[...]
        m = jnp.max(logits, axis=1, keepdims=True)
        lse = jnp.log(jnp.sum(jnp.exp(logits - m), axis=1, keepdims=True)) + m
        o_ref[...] = logits - lse


def _final(s, hp, inv, b, wcp, bcp):
    return pl.pallas_call(
        _fin_body,
        grid=(GRID,),
        in_specs=[
            pl.BlockSpec((NC, BLK, D), lambda i: (0, i, 0)),
            pl.BlockSpec((BLK, D), lambda i: (i, 0)),
            pl.BlockSpec((BLK, 1), lambda i: (i, 0)),
            pl.BlockSpec((1, D), lambda i: (0, 0)),
            pl.BlockSpec((D, D), lambda i: (0, 0)),
            pl.BlockSpec((1, D), lambda i: (0, 0)),
        ],
        out_specs=pl.BlockSpec((1, D), lambda i: (0, 0)),
        out_shape=jax.ShapeDtypeStruct((1, D), jnp.float32),
        scratch_shapes=[pltpu.VMEM((1, D), jnp.float32)],
    )(s, hp, inv, b, wcp, bcp)


def kernel(x, edge_index, W1, b1, W2, b2, Wc, bc):
    src = edge_index[0].astype(jnp.int32)
    dst = edge_index[1].astype(jnp.int32)
    pad = E_PAD - E
    srcp = jnp.concatenate([src, jnp.full((pad,), N, jnp.int32)])
    dstp = jnp.concatenate([dst, jnp.full((pad,), N, jnp.int32)])
    srcp = srcp.reshape(NTILES, NCHUNK, CHUNK)
    dstp = dstp.reshape(NTILES * NCHUNK, CHUNK)

    degp = _deg_kernel(dstp).reshape(NC, N_PAD, L)

    xp = jnp.pad(x, ((0, N_PAD - N), (0, 0)))
    u1 = _matmul(xp, W1)
    h1p, inv = _scale(degp, u1)

    s1 = _agg_kernel(h1p, srcp, dstp).reshape(NC, N_PAD, D)

    h2p = _mid(s1, h1p, inv, b1.reshape(1, D), W2)

    s2 = _agg_kernel(h2p, srcp, dstp).reshape(NC, N_PAD, D)

    wcp = jnp.pad(Wc, ((0, 0), (0, D - Wc.shape[1])))
    bcp = jnp.concatenate([bc, jnp.full((D - bc.shape[0],), -1e30,
                                        jnp.float32)]).reshape(1, D)
    out = _final(s2, h2p, inv, b2.reshape(1, D), wcp, bcp)
    return out[0, :2]


# speedup vs baseline: 8.9879x; 1.1949x over previous
import functools

import jax
import jax.numpy as jnp
from jax import lax
from jax.experimental import pallas as pl
from jax.experimental.pallas import tpu as pltpu
from jax.experimental.pallas import tpu_sc as plsc

N = 10000
E = 320000
D = 128
NC = 2
NS = 16
L = 16
NTILES = NC * NS
CHUNK = 128
NCHUNK = 80
EPT = CHUNK * NCHUNK
E_PAD = EPT * NTILES
N_PAD = 10240
RPT = N_PAD // NS

BLK = 640
GRID = N_PAD // BLK


def _mesh():
    return plsc.VectorSubcoreMesh(core_axis_name="c", subcore_axis_name="s")


@functools.partial(
    pl.kernel,
    out_type=jax.ShapeDtypeStruct((NC * N_PAD, L), jnp.float32),
    mesh=_mesh(),
    scratch_types=[
        pltpu.VMEM((CHUNK,), jnp.int32),
        pltpu.VMEM((CHUNK,), jnp.int32),
        pltpu.VMEM((CHUNK, L), jnp.float32),
        pltpu.VMEM((CHUNK, L), jnp.float32),
        pltpu.VMEM_SHARED((N_PAD, L), jnp.float32),
        pltpu.SemaphoreType.DMA,
        pltpu.SemaphoreType.DMA,
    ],
)
def _deg_kernel(dst_hbm, out_hbm, dcur0_v, dcur1_v, ones_v, zeros_v, acc_sh,
                isem0, isem1):
    c = lax.axis_index("c")
    s = lax.axis_index("s")
    w = c * NS + s

    def fill(i, _):
        ones_v[i, :] = jnp.ones((L,), jnp.float32)
        zeros_v[i, :] = jnp.zeros((L,), jnp.float32)
        return 0

    lax.fori_loop(0, CHUNK, fill, 0)

    zbase = s * RPT
    for b in range(RPT // CHUNK):
        pltpu.sync_copy(zeros_v, acc_sh.at[pl.ds(zbase + b * CHUNK, CHUNK)])
    plsc.subcore_barrier()

    pltpu.async_copy(dst_hbm.at[w * NCHUNK], dcur0_v, isem0)

    def body(jj, _):
        j = 2 * jj
        pltpu.async_copy(dst_hbm.at[w * NCHUNK + j + 1], dcur1_v, isem1)
        pltpu.make_async_copy(dst_hbm.at[0], dcur0_v, isem0).wait()
        pltpu.sync_copy(ones_v, acc_sh.at[dcur0_v], add=True)

        @pl.when(jj < NCHUNK // 2 - 1)
        def _():
            pltpu.async_copy(dst_hbm.at[w * NCHUNK + j + 2], dcur0_v, isem0)

        pltpu.make_async_copy(dst_hbm.at[0], dcur1_v, isem1).wait()
        pltpu.sync_copy(ones_v, acc_sh.at[dcur1_v], add=True)
        return 0

    lax.fori_loop(0, NCHUNK // 2, body, 0)
    plsc.subcore_barrier()

    pltpu.sync_copy(
        acc_sh.at[pl.ds(s * RPT, RPT)],
        out_hbm.at[pl.ds(c * N_PAD + s * RPT, RPT)],
    )


@functools.partial(
    pl.kernel,
    out_type=jax.ShapeDtypeStruct((NC * N_PAD, D), jnp.float32),
    mesh=_mesh(),
    scratch_types=[
        pltpu.VMEM((NCHUNK, CHUNK), jnp.int32),
        pltpu.VMEM((CHUNK,), jnp.int32),
        pltpu.VMEM((CHUNK,), jnp.int32),
        pltpu.VMEM((CHUNK, D), jnp.float32),
        pltpu.VMEM((CHUNK, D), jnp.float32),
        pltpu.VMEM_SHARED((N_PAD, D), jnp.float32),
        pltpu.SemaphoreType.DMA,
        pltpu.SemaphoreType.DMA,
        pltpu.SemaphoreType.DMA,
        pltpu.SemaphoreType.DMA,
    ],
)
def _agg_kernel(hp_hbm, src_hbm, dst_hbm, out_hbm, src_v, dcur0_v, dcur1_v,
                rows0_v, rows1_v, acc_sh, gsem0, gsem1, isem0, isem1):
    c = lax.axis_index("c")
    s = lax.axis_index("s")
    w = c * NS + s

    def zr(i, _):
        for jj in range(D // L):
            rows0_v[i, pl.ds(jj * L, L)] = jnp.zeros((L,), jnp.float32)
        return 0

    lax.fori_loop(0, CHUNK, zr, 0)

    zbase = s * RPT
    for b in range(RPT // CHUNK):
        pltpu.sync_copy(rows0_v, acc_sh.at[pl.ds(zbase + b * CHUNK, CHUNK)])
    plsc.subcore_barrier()

    pltpu.sync_copy(src_hbm.at[w], src_v)

    pltpu.async_copy(hp_hbm.at[src_v.at[0]], rows0_v, gsem0)
    pltpu.async_copy(dst_hbm.at[w * NCHUNK], dcur0_v, isem0)

    def body(jj, _):
        j = 2 * jj
        pltpu.async_copy(hp_hbm.at[src_v.at[j + 1]], rows1_v, gsem1)
        pltpu.async_copy(dst_hbm.at[w * NCHUNK + j + 1], dcur1_v, isem1)
        pltpu.make_async_copy(hp_hbm.at[src_v.at[j]], rows0_v, gsem0).wait()
        pltpu.make_async_copy(dst_hbm.at[0], dcur0_v, isem0).wait()
        pltpu.sync_copy(rows0_v, acc_sh.at[dcur0_v], add=True)

        @pl.when(jj < NCHUNK // 2 - 1)
        def _():
            pltpu.async_copy(hp_hbm.at[src_v.at[j + 2]], rows0_v, gsem0)
            pltpu.async_copy(dst_hbm.at[w * NCHUNK + j + 2], dcur0_v, isem0)

        pltpu.make_async_copy(hp_hbm.at[src_v.at[j + 1]], rows1_v, gsem1).wait()
        pltpu.make_async_copy(dst_hbm.at[0], dcur1_v, isem1).wait()
        pltpu.sync_copy(rows1_v, acc_sh.at[dcur1_v], add=True)
        return 0

    lax.fori_loop(0, NCHUNK // 2, body, 0)
    plsc.subcore_barrier()

    pltpu.sync_copy(
        acc_sh.at[pl.ds(s * RPT, RPT)],
        out_hbm.at[pl.ds(c * N_PAD + s * RPT, RPT)],
    )


def _mm_body(x_ref, w_ref, o_ref):
    o_ref[...] = jnp.dot(x_ref[...], w_ref[...],
                         preferred_element_type=jnp.float32)


def _matmul(x, w):
    return pl.pallas_call(
        _mm_body,
        grid=(GRID,),
        in_specs=[
            pl.BlockSpec((BLK, D), lambda i: (i, 0)),
            pl.BlockSpec((D, D), lambda i: (0, 0)),
        ],
        out_specs=pl.BlockSpec((BLK, D), lambda i: (i, 0)),
        out_shape=jax.ShapeDtypeStruct((N_PAD, D), jnp.float32),
    )(x, w)


def _scale_body(d_ref, u_ref, hp_ref, inv_ref):
    cnt = d_ref[0, :, 0:1] + d_ref[1, :, 0:1]
    inv = lax.rsqrt(cnt + 1.0)
    inv_ref[...] = inv
    hp_ref[...] = u_ref[...] * inv


def _scale(degp, u):
    return pl.pallas_call(
        _scale_body,
        grid=(GRID,),
        in_specs=[
            pl.BlockSpec((NC, BLK, L), lambda i: (0, i, 0)),
            pl.BlockSpec((BLK, D), lambda i: (i, 0)),
        ],
        out_specs=[
            pl.BlockSpec((BLK, D), lambda i: (i, 0)),
            pl.BlockSpec((BLK, 1), lambda i: (i, 0)),
        ],
        out_shape=[
            jax.ShapeDtypeStruct((N_PAD, D), jnp.float32),
            jax.ShapeDtypeStruct((N_PAD, 1), jnp.float32),
        ],
    )(degp, u)


def _row_mask(i):
    ridx = lax.broadcasted_iota(jnp.int32, (BLK, 1), 0) + i * BLK
    return ridx < N


def _mid_body(s_ref, hp_ref, inv_ref, b_ref, w_ref, o_ref):
    inv = inv_ref[...]
    t = (s_ref[0] + s_ref[1] + hp_ref[...]) * inv + b_ref[...]
    h = jnp.maximum(t, 0.0)
    h = jnp.where(_row_mask(pl.program_id(0)), h, 0.0)
    o_ref[...] = jnp.dot(h, w_ref[...],
                         preferred_element_type=jnp.float32) * inv


def _mid(s, hp, inv, b, w):
    return pl.pallas_call(
        _mid_body,
        grid=(GRID,),
        in_specs=[
            pl.BlockSpec((NC, BLK, D), lambda i: (0, i, 0)),
            pl.BlockSpec((BLK, D), lambda i: (i, 0)),
            pl.BlockSpec((BLK, 1), lambda i: (i, 0)),
            pl.BlockSpec((1, D), lambda i: (0, 0)),
            pl.BlockSpec((D, D), lambda i: (0, 0)),
        ],
        out_specs=pl.BlockSpec((BLK, D), lambda i: (i, 0)),
        out_shape=jax.ShapeDtypeStruct((N_PAD, D), jnp.float32),
    )(s, hp, inv, b, w)


def _fin_body(s_ref, hp_ref, inv_ref, b_ref, wc_ref, bc_ref, o_ref, acc_ref):
    i = pl.program_id(0)

    @pl.when(i == 0)
    def _():
        acc_ref[...] = jnp.zeros_like(acc_ref)

    t = (s_ref[0] + s_ref[1] + hp_ref[...]) * inv_ref[...] + b_ref[...]
    h = jnp.maximum(t, 0.0)
    h = jnp.where(_row_mask(i), h, 0.0)
    acc_ref[...] += jnp.sum(h, axis=0, keepdims=True)

    @pl.when(i == pl.num_programs(0) - 1)
    def _():
        g = acc_ref[...] * (1.0 / N)
        logits = jnp.dot(g, wc_ref[...],
                         preferred_element_type=jnp.float32) + bc_ref[...]
        m = jnp.max(logits, axis=1, keepdims=True)
        lse = jnp.log(jnp.sum(jnp.exp(logits - m), axis=1, keepdims=True)) + m
        o_ref[...] = logits - lse


def _final(s, hp, inv, b, wcp, bcp):
    return pl.pallas_call(
        _fin_body,
        grid=(GRID,),
        in_specs=[
            pl.BlockSpec((NC, BLK, D), lambda i: (0, i, 0)),
            pl.BlockSpec((BLK, D), lambda i: (i, 0)),
            pl.BlockSpec((BLK, 1), lambda i: (i, 0)),
            pl.BlockSpec((1, D), lambda i: (0, 0)),
            pl.BlockSpec((D, D), lambda i: (0, 0)),
            pl.BlockSpec((1, D), lambda i: (0, 0)),
        ],
        out_specs=pl.BlockSpec((1, D), lambda i: (0, 0)),
        out_shape=jax.ShapeDtypeStruct((1, D), jnp.float32),
        scratch_shapes=[pltpu.VMEM((1, D), jnp.float32)],
    )(s, hp, inv, b, wcp, bcp)


def kernel(x, edge_index, W1, b1, W2, b2, Wc, bc):
    src = edge_index[0].astype(jnp.int32)
    dst = edge_index[1].astype(jnp.int32)
    pad = E_PAD - E
    srcp = jnp.concatenate([src, jnp.full((pad,), N, jnp.int32)])
    dstp = jnp.concatenate([dst, jnp.full((pad,), N, jnp.int32)])
    srcp = srcp.reshape(NTILES, NCHUNK, CHUNK)
    dstp = dstp.reshape(NTILES * NCHUNK, CHUNK)

    degp = _deg_kernel(dstp).reshape(NC, N_PAD, L)

    xp = jnp.pad(x, ((0, N_PAD - N), (0, 0)))
    u1 = _matmul(xp, W1)
    h1p, inv = _scale(degp, u1)

    s1 = _agg_kernel(h1p, srcp, dstp).reshape(NC, N_PAD, D)

    h2p = _mid(s1, h1p, inv, b1.reshape(1, D), W2)

    s2 = _agg_kernel(h2p, srcp, dstp).reshape(NC, N_PAD, D)

    wcp = jnp.pad(Wc, ((0, 0), (0, D - Wc.shape[1])))
    bcp = jnp.concatenate([bc, jnp.full((D - bc.shape[0],), -1e30,
                                        jnp.float32)]).reshape(1, D)
    out = _final(s2, h2p, inv, b2.reshape(1, D), wcp, bcp)
    return out[0, :2]


# speedup vs baseline: 9.5883x; 1.0668x over previous
import functools

import jax
import jax.numpy as jnp
from jax import lax
from jax.experimental import pallas as pl
from jax.experimental.pallas import tpu as pltpu
from jax.experimental.pallas import tpu_sc as plsc

N = 10000
E = 320000
D = 128
NC = 2
NS = 16
L = 16
NTILES = NC * NS
CHUNK = 64
NCHUNK = 160
NBUF = 4
EPT = CHUNK * NCHUNK
E_PAD = EPT * NTILES
N_PAD = 10240
RPT = N_PAD // NS

BLK = 640
GRID = N_PAD // BLK


CHUNKD = 128
NCHUNKD = EPT // CHUNKD


def _mesh():
    return plsc.VectorSubcoreMesh(core_axis_name="c", subcore_axis_name="s")


@functools.partial(
    pl.kernel,
    out_type=jax.ShapeDtypeStruct((NC * N_PAD, L), jnp.float32),
    mesh=_mesh(),
    scratch_types=[
        pltpu.VMEM((CHUNKD,), jnp.int32),
        pltpu.VMEM((CHUNKD,), jnp.int32),
        pltpu.VMEM((CHUNKD, L), jnp.float32),
        pltpu.VMEM((CHUNKD, L), jnp.float32),
        pltpu.VMEM_SHARED((N_PAD, L), jnp.float32),
        pltpu.SemaphoreType.DMA,
        pltpu.SemaphoreType.DMA,
    ],
)
def _deg_kernel(dst_hbm, out_hbm, dcur0_v, dcur1_v, ones_v, zeros_v, acc_sh,
                isem0, isem1):
    c = lax.axis_index("c")
    s = lax.axis_index("s")
    w = c * NS + s

    def fill(i, _):
        ones_v[i, :] = jnp.ones((L,), jnp.float32)
        zeros_v[i, :] = jnp.zeros((L,), jnp.float32)
        return 0

    lax.fori_loop(0, CHUNKD, fill, 0)

    zbase = s * RPT
    for b in range(RPT // CHUNKD):
        pltpu.sync_copy(zeros_v, acc_sh.at[pl.ds(zbase + b * CHUNKD, CHUNKD)])
    plsc.subcore_barrier()

    pltpu.async_copy(dst_hbm.at[w * NCHUNKD], dcur0_v, isem0)

    def body(jj, _):
        j = 2 * jj
        pltpu.async_copy(dst_hbm.at[w * NCHUNKD + j + 1], dcur1_v, isem1)
        pltpu.make_async_copy(dst_hbm.at[0], dcur0_v, isem0).wait()
        pltpu.sync_copy(ones_v, acc_sh.at[dcur0_v], add=True)

        @pl.when(jj < NCHUNKD // 2 - 1)
        def _():
            pltpu.async_copy(dst_hbm.at[w * NCHUNKD + j + 2], dcur0_v, isem0)

        pltpu.make_async_copy(dst_hbm.at[0], dcur1_v, isem1).wait()
        pltpu.sync_copy(ones_v, acc_sh.at[dcur1_v], add=True)
        return 0

    lax.fori_loop(0, NCHUNKD // 2, body, 0)
    plsc.subcore_barrier()

    pltpu.sync_copy(
        acc_sh.at[pl.ds(s * RPT, RPT)],
        out_hbm.at[pl.ds(c * N_PAD + s * RPT, RPT)],
    )


def _sidx(src_v, t, pp):
    return src_v.at[2 * t + pp // 2, pl.ds((pp % 2) * CHUNK, CHUNK)]


@functools.partial(
    pl.kernel,
    out_type=jax.ShapeDtypeStruct((NC * N_PAD, D), jnp.float32),
    mesh=_mesh(),
    scratch_types=(
        [pltpu.VMEM((NCHUNK // 2, 2 * CHUNK), jnp.int32)]
        + [pltpu.VMEM((CHUNK,), jnp.int32) for _ in range(NBUF)]
        + [pltpu.VMEM((CHUNK, D), jnp.float32) for _ in range(NBUF)]
        + [pltpu.VMEM_SHARED((N_PAD, D), jnp.float32)]
        + [pltpu.SemaphoreType.DMA for _ in range(3 * NBUF)]
    ),
)
def _agg_kernel(hp_hbm, src_hbm, dst_hbm, out_hbm, src_v, *scr):
    ibuf = scr[0:NBUF]
    rows = scr[NBUF:2 * NBUF]
    acc_sh = scr[2 * NBUF]
    gsem = scr[2 * NBUF + 1:3 * NBUF + 1]
    isem = scr[3 * NBUF + 1:4 * NBUF + 1]
    ssem = scr[4 * NBUF + 1:5 * NBUF + 1]
    c = lax.axis_index("c")
    sc = lax.axis_index("s")
    w = c * NS + sc

    def zr(i, _):
        for jj in range(D // L):
            rows[0][i, pl.ds(jj * L, L)] = jnp.zeros((L,), jnp.float32)
        return 0

    lax.fori_loop(0, CHUNK, zr, 0)

    zbase = sc * RPT
    for b in range(RPT // CHUNK):
        pltpu.sync_copy(rows[0], acc_sh.at[pl.ds(zbase + b * CHUNK, CHUNK)])
    plsc.subcore_barrier()

    pltpu.sync_copy(src_hbm.at[w], src_v)

    for p in range(2):
        pltpu.async_copy(hp_hbm.at[src_v.at[0, pl.ds(p * CHUNK, CHUNK)]],
                         rows[p], gsem[p])
        pltpu.async_copy(dst_hbm.at[w * NCHUNK + p], ibuf[p], isem[p])

    def body(t, _):
        for p in range(NBUF):
            j = t * NBUF + p
            q = (p + 2) % NBUF
            pltpu.make_async_copy(hp_hbm.at[_sidx(src_v, t, p)], rows[p],
                                  gsem[p]).wait()
            pltpu.make_async_copy(dst_hbm.at[0], ibuf[p], isem[p]).wait()
            pltpu.sync_copy(rows[p], acc_sh.at[ibuf[p]], add=True)

            @pl.when(j + 2 < NCHUNK)
            def _():
                pltpu.async_copy(hp_hbm.at[_sidx(src_v, t, p + 2)],
                                 rows[q], gsem[q])
                pltpu.async_copy(dst_hbm.at[w * NCHUNK + j + 2], ibuf[q],
                                 isem[q])
        return 0

    lax.fori_loop(0, NCHUNK // NBUF, body, 0)
    plsc.subcore_barrier()

    pltpu.sync_copy(
        acc_sh.at[pl.ds(sc * RPT, RPT)],
        out_hbm.at[pl.ds(c * N_PAD + sc * RPT, RPT)],
    )


def _mm_body(x_ref, w_ref, o_ref):
    o_ref[...] = jnp.dot(x_ref[...], w_ref[...],
                         preferred_element_type=jnp.float32)


def _matmul(x, w):
    return pl.pallas_call(
        _mm_body,
        grid=(GRID,),
        in_specs=[
            pl.BlockSpec((BLK, D), lambda i: (i, 0)),
            pl.BlockSpec((D, D), lambda i: (0, 0)),
        ],
        out_specs=pl.BlockSpec((BLK, D), lambda i: (i, 0)),
        out_shape=jax.ShapeDtypeStruct((N_PAD, D), jnp.float32),
    )(x, w)


def _scale_body(d_ref, u_ref, hp_ref, inv_ref):
    cnt = d_ref[0, :, 0:1] + d_ref[1, :, 0:1]
    inv = lax.rsqrt(cnt + 1.0)
    inv_ref[...] = inv
    hp_ref[...] = u_ref[...] * inv


def _scale(degp, u):
    return pl.pallas_call(
        _scale_body,
        grid=(GRID,),
        in_specs=[
            pl.BlockSpec((NC, BLK, L), lambda i: (0, i, 0)),
            pl.BlockSpec((BLK, D), lambda i: (i, 0)),
        ],
        out_specs=[
            pl.BlockSpec((BLK, D), lambda i: (i, 0)),
            pl.BlockSpec((BLK, 1), lambda i: (i, 0)),
        ],
        out_shape=[
            jax.ShapeDtypeStruct((N_PAD, D), jnp.float32),
            jax.ShapeDtypeStruct((N_PAD, 1), jnp.float32),
        ],
    )(degp, u)


def _row_mask(i):
    ridx = lax.broadcasted_iota(jnp.int32, (BLK, 1), 0) + i * BLK
    return ridx < N


def _mid_body(s_ref, hp_ref, inv_ref, b_ref, w_ref, o_ref):
    inv = inv_ref[...]
    t = (s_ref[0] + s_ref[1] + hp_ref[...]) * inv + b_ref[...]
    h = jnp.maximum(t, 0.0)
    h = jnp.where(_row_mask(pl.program_id(0)), h, 0.0)
    o_ref[...] = jnp.dot(h, w_ref[...],
                         preferred_element_type=jnp.float32) * inv


def _mid(s, hp, inv, b, w):
    return pl.pallas_call(
        _mid_body,
        grid=(GRID,),
        in_specs=[
            pl.BlockSpec((NC, BLK, D), lambda i: (0, i, 0)),
            pl.BlockSpec((BLK, D), lambda i: (i, 0)),
            pl.BlockSpec((BLK, 1), lambda i: (i, 0)),
            pl.BlockSpec((1, D), lambda i: (0, 0)),
            pl.BlockSpec((D, D), lambda i: (0, 0)),
        ],
        out_specs=pl.BlockSpec((BLK, D), lambda i: (i, 0)),
        out_shape=jax.ShapeDtypeStruct((N_PAD, D), jnp.float32),
    )(s, hp, inv, b, w)


def _fin_body(s_ref, hp_ref, inv_ref, b_ref, wc_ref, bc_ref, o_ref, acc_ref):
    i = pl.program_id(0)

    @pl.when(i == 0)
    def _():
        acc_ref[...] = jnp.zeros_like(acc_ref)

    t = (s_ref[0] + s_ref[1] + hp_ref[...]) * inv_ref[...] + b_ref[...]
    h = jnp.maximum(t, 0.0)
    h = jnp.where(_row_mask(i), h, 0.0)
    acc_ref[...] += jnp.sum(h, axis=0, keepdims=True)

    @pl.when(i == pl.num_programs(0) - 1)
    def _():
        g = acc_ref[...] * (1.0 / N)
        logits = jnp.dot(g, wc_ref[...],
                         preferred_element_type=jnp.float32) + bc_ref[...]
        m = jnp.max(logits, axis=1, keepdims=True)
        lse = jnp.log(jnp.sum(jnp.exp(logits - m), axis=1, keepdims=True)) + m
        o_ref[...] = logits - lse


def _final(s, hp, inv, b, wcp, bcp):
    return pl.pallas_call(
        _fin_body,
        grid=(GRID,),
        in_specs=[
            pl.BlockSpec((NC, BLK, D), lambda i: (0, i, 0)),
            pl.BlockSpec((BLK, D), lambda i: (i, 0)),
            pl.BlockSpec((BLK, 1), lambda i: (i, 0)),
            pl.BlockSpec((1, D), lambda i: (0, 0)),
            pl.BlockSpec((D, D), lambda i: (0, 0)),
            pl.BlockSpec((1, D), lambda i: (0, 0)),
        ],
        out_specs=pl.BlockSpec((1, D), lambda i: (0, 0)),
        out_shape=jax.ShapeDtypeStruct((1, D), jnp.float32),
        scratch_shapes=[pltpu.VMEM((1, D), jnp.float32)],
    )(s, hp, inv, b, wcp, bcp)


def kernel(x, edge_index, W1, b1, W2, b2, Wc, bc):
    src = edge_index[0].astype(jnp.int32)
    dst = edge_index[1].astype(jnp.int32)
    pad = E_PAD - E
    srcp = jnp.concatenate([src, jnp.full((pad,), N, jnp.int32)])
    dstp = jnp.concatenate([dst, jnp.full((pad,), N, jnp.int32)])
    srcp = srcp.reshape(NTILES, NCHUNK // 2, 2 * CHUNK)
    dstp_d = dstp.reshape(NTILES * NCHUNKD, CHUNKD)
    dstp = dstp.reshape(NTILES * NCHUNK, CHUNK)

    degp = _deg_kernel(dstp_d).reshape(NC, N_PAD, L)

    xp = jnp.pad(x, ((0, N_PAD - N), (0, 0)))
    u1 = _matmul(xp, W1)
    h1p, inv = _scale(degp, u1)

    s1 = _agg_kernel(h1p, srcp, dstp).reshape(NC, N_PAD, D)

    h2p = _mid(s1, h1p, inv, b1.reshape(1, D), W2)

    s2 = _agg_kernel(h2p, srcp, dstp).reshape(NC, N_PAD, D)

    wcp = jnp.pad(Wc, ((0, 0), (0, D - Wc.shape[1])))
    bcp = jnp.concatenate([bc, jnp.full((D - bc.shape[0],), -1e30,
                                        jnp.float32)]).reshape(1, D)
    out = _final(s2, h2p, inv, b2.reshape(1, D), wcp, bcp)
    return out[0, :2]
